# baseline (speedup 1.0000x reference)
import functools
import jax
import jax.numpy as jnp
from jax import lax
from jax.experimental import pallas as pl
from jax.experimental.pallas import tpu as pltpu
from jax.experimental.pallas import tpu_sc as plsc

_LAMDA = 0.5
_BLK = 512
_CH = 16
_SROWS = 1024


def _sc_segment_sum(feat, target):
    b, f = feat.shape
    info = plsc.get_sparse_core_info()
    nc, ns = info.num_cores, info.num_subcores
    nw = nc * ns
    rows_per_tile = b // nw
    nch = rows_per_tile // _CH
    srows = _SROWS // ns
    mesh = plsc.VectorSubcoreMesh(core_axis_name="c", subcore_axis_name="s")

    @functools.partial(
        pl.kernel,
        mesh=mesh,
        out_type=jax.ShapeDtypeStruct((nc, _SROWS, f), jnp.float32),
        scratch_types=[
            pltpu.VMEM((_CH,), jnp.int32),
            pltpu.VMEM((_CH, f), jnp.float32),
            pltpu.VMEM((16, f), jnp.float32),
            pltpu.VMEM_SHARED((_SROWS, f), jnp.float32),
        ],
        compiler_params=pltpu.CompilerParams(use_tc_tiling_on_sc=False),
    )
    def sc_kernel(feat_hbm, tgt_hbm, out_hbm, idx_v, fbuf, zbuf, s_acc):
        c = lax.axis_index("c")
        s = lax.axis_index("s")
        wid = s * nc + c

        z = jnp.zeros((16,), jnp.float32)

        def zrow(r, carry):
            for j in range(f // 16):
                zbuf[r, pl.ds(j * 16, 16)] = z
            zbuf[r, pl.ds(f - 16, 16)] = z
            return carry

        lax.fori_loop(0, 16, zrow, 0)

        for t in range(srows // 16):
            pltpu.sync_copy(zbuf, s_acc.at[pl.ds(s * srows + t * 16, 16)])
        plsc.subcore_barrier()

        base0 = wid * rows_per_tile

        def chunk(i, carry):
            base = base0 + i * _CH
            pltpu.sync_copy(tgt_hbm.at[pl.ds(base, _CH)], idx_v)
            pltpu.sync_copy(feat_hbm.at[pl.ds(base, _CH)], fbuf)
            pltpu.sync_copy(fbuf, s_acc.at[idx_v], add=True)
            return carry

        lax.fori_loop(0, nch, chunk, 0)
        plsc.subcore_barrier()

        for t in range(srows // 16):
            lo = s * srows + t * 16
            pltpu.sync_copy(s_acc.at[pl.ds(lo, 16)], out_hbm.at[c, pl.ds(lo, 16)])

    return sc_kernel(feat, target)


def _pass1_kernel(tgt_ref, x_ref, out_ref, counts_ref, acc_ref, *, nblk):
    i = pl.program_id(0)

    @pl.when(i == 0)
    def _init():
        acc_ref[0, 0] = 0.0
        acc_ref[0, 1] = 0.0
        counts_ref[...] = jnp.zeros_like(counts_ref)

    x = x_ref[...]
    tgt = tgt_ref[0, 0, :]
    blk, f = x.shape

    m = jnp.max(x, axis=1, keepdims=True)
    lse = jnp.log(jnp.sum(jnp.exp(x - m), axis=1, keepdims=True)) + m

    cols = jax.lax.broadcasted_iota(jnp.int32, (blk, f), 1)
    mask = cols == tgt[:, None]
    picked_sum = jnp.sum(jnp.where(mask, x, 0.0))
    counts_ref[...] += jnp.sum(mask.astype(jnp.float32), axis=0, keepdims=True)

    acc_ref[0, 0] += jnp.sum(lse) - picked_sum
    acc_ref[0, 1] += jnp.sum(x * x)

    @pl.when(i == nblk - 1)
    def _fin():
        out_ref[0, 0] = acc_ref[0, 0]
        out_ref[0, 1] = acc_ref[0, 1]


def _tc_pass1(feat, target):
    batch, f = feat.shape
    nblk = batch // _BLK
    tgt3 = target.reshape(nblk, 1, _BLK)
    return pl.pallas_call(
        functools.partial(_pass1_kernel, nblk=nblk),
        grid=(nblk,),
        in_specs=[
            pl.BlockSpec((1, 1, _BLK), lambda i: (i, 0, 0)),
            pl.BlockSpec((_BLK, f), lambda i: (i, 0)),
        ],
        out_specs=[
            pl.BlockSpec(memory_space=pltpu.SMEM),
            pl.BlockSpec((1, f), lambda i: (0, 0)),
        ],
        out_shape=[
            jax.ShapeDtypeStruct((1, 2), jnp.float32),
            jax.ShapeDtypeStruct((1, f), jnp.float32),
        ],
        scratch_shapes=[pltpu.SMEM((1, 2), jnp.float32)],
    )(tgt3, feat)


def _combine_kernel(cen_ref, s_ref, counts_ref, scal_ref, out_ref, *, batch):
    cen = cen_ref[...]
    c = cen.shape[0]
    s_sum = s_ref[0] + s_ref[1]
    dot_sum = jnp.sum(cen * s_sum[:c, :])
    c2 = jnp.sum(cen * cen, axis=1, keepdims=True)
    cterm = jax.lax.dot_general(
        counts_ref[...], c2,
        (((1,), (0,)), ((), ())),
        precision=jax.lax.Precision.HIGHEST,
        preferred_element_type=jnp.float32,
    )[0, 0]
    soft = scal_ref[0, 0]
    f2 = scal_ref[0, 1]
    center = cterm - 2.0 * dot_sum + f2
    out_ref[0, 0] = soft / batch + _LAMDA * center / 2.0 / batch


def _tc_combine(centers, s_partials, counts, scalars, batch):
    c, f = centers.shape
    return pl.pallas_call(
        functools.partial(_combine_kernel, batch=batch),
        in_specs=[
            pl.BlockSpec((c, f), lambda: (0, 0)),
            pl.BlockSpec((2, _SROWS, f), lambda: (0, 0, 0)),
            pl.BlockSpec((1, c), lambda: (0, 0)),
            pl.BlockSpec(memory_space=pltpu.SMEM),
        ],
        out_specs=pl.BlockSpec(memory_space=pltpu.SMEM),
        out_shape=jax.ShapeDtypeStruct((1, 1), jnp.float32),
    )(centers, s_partials, counts, scalars)


def kernel(feat, target, centers):
    batch = feat.shape[0]
    tgt = target.astype(jnp.int32)
    s_partials = _sc_segment_sum(feat, tgt)
    scalars, counts = _tc_pass1(feat, tgt)
    out = _tc_combine(centers, s_partials, counts, scalars, batch)
    return out[0, 0]


# speedup vs baseline: 1.1650x; 1.1650x over previous
import functools
import jax
import jax.numpy as jnp
from jax import lax
from jax.experimental import pallas as pl
from jax.experimental.pallas import tpu as pltpu
from jax.experimental.pallas import tpu_sc as plsc

_LAMDA = 0.5
_BLK = 512
_CH = 16
_SROWS = 1024


def _sc_segment_sum(feat, target):
    b, f = feat.shape
    info = plsc.get_sparse_core_info()
    nc, ns = info.num_cores, info.num_subcores
    nw = nc * ns
    rows_per_tile = b // nw
    nch = rows_per_tile // _CH
    nb = 3
    srows = _SROWS // ns
    tgt2 = target.reshape(b // _CH, _CH)
    mesh = plsc.VectorSubcoreMesh(core_axis_name="c", subcore_axis_name="s")

    @functools.partial(
        pl.kernel,
        mesh=mesh,
        out_type=jax.ShapeDtypeStruct((nc, _SROWS, f), jnp.float32),
        scratch_types=[
            pltpu.VMEM((nch, _CH), jnp.int32),
            pltpu.VMEM((nb, _CH, f), jnp.float32),
            pltpu.VMEM((8, f), jnp.float32),
            pltpu.VMEM_SHARED((_SROWS, f), jnp.float32),
        ] + [pltpu.SemaphoreType.DMA] * 6,
        compiler_params=pltpu.CompilerParams(use_tc_tiling_on_sc=False),
    )
    def sc_kernel(feat_hbm, tgt_hbm, out_hbm, idx_v, fbuf, zbuf, s_acc,
                  *sems):
        c = lax.axis_index("c")
        s = lax.axis_index("s")
        wid = s * nc + c

        pltpu.sync_copy(tgt_hbm.at[pl.ds(wid * nch, nch)], idx_v)

        z = jnp.zeros((16,), jnp.float32)

        def zrow(r, carry):
            for j in range(f // 16):
                zbuf[r, pl.ds(j * 16, 16)] = z
            zbuf[r, pl.ds(f - 16, 16)] = z
            return carry

        lax.fori_loop(0, 8, zrow, 0)

        for t in range(srows // 8):
            pltpu.sync_copy(zbuf, s_acc.at[pl.ds(s * srows + t * 8, 8)])
        plsc.subcore_barrier()

        base0 = wid * rows_per_tile
        ld_sems = list(sems[:nb])
        st_sems = list(sems[nb:])
        ld_descs = [None] * nb
        st_descs = [None] * nb

        for j in range(nch + 1):
            bj = j % nb
            if j < nch:
                if j >= nb:
                    st_descs[bj].wait()
                ld_descs[bj] = pltpu.async_copy(
                    feat_hbm.at[pl.ds(base0 + j * _CH, _CH)],
                    fbuf.at[bj], ld_sems[bj])
            if j >= 1:
                pb = (j - 1) % nb
                ld_descs[pb].wait()
                st_descs[pb] = pltpu.async_copy(
                    fbuf.at[pb], s_acc.at[idx_v.at[j - 1]],
                    st_sems[pb], add=True)
        for bj in range(nb):
            st_descs[(nch - 1 - bj) % nb].wait()
        plsc.subcore_barrier()

        for t in range(srows // 16):
            lo = s * srows + t * 16
            pltpu.sync_copy(s_acc.at[pl.ds(lo, 16)], out_hbm.at[c, pl.ds(lo, 16)])

    return sc_kernel(feat, tgt2)


def _pass1_kernel(tgt_ref, x_ref, out_ref, counts_ref, acc_ref, *, nblk):
    i = pl.program_id(0)

    @pl.when(i == 0)
    def _init():
        acc_ref[0, 0] = 0.0
        acc_ref[0, 1] = 0.0
        counts_ref[...] = jnp.zeros_like(counts_ref)

    x = x_ref[...]
    tgt = tgt_ref[0, 0, :]
    blk, f = x.shape

    m = jnp.max(x, axis=1, keepdims=True)
    lse = jnp.log(jnp.sum(jnp.exp(x - m), axis=1, keepdims=True)) + m

    cols = jax.lax.broadcasted_iota(jnp.int32, (blk, f), 1)
    mask = cols == tgt[:, None]
    picked_sum = jnp.sum(jnp.where(mask, x, 0.0))
    counts_ref[...] += jnp.sum(mask.astype(jnp.float32), axis=0, keepdims=True)

    acc_ref[0, 0] += jnp.sum(lse) - picked_sum
    acc_ref[0, 1] += jnp.sum(x * x)

    @pl.when(i == nblk - 1)
    def _fin():
        out_ref[0, 0] = acc_ref[0, 0]
        out_ref[0, 1] = acc_ref[0, 1]


def _tc_pass1(feat, target):
    batch, f = feat.shape
    nblk = batch // _BLK
    tgt3 = target.reshape(nblk, 1, _BLK)
    return pl.pallas_call(
        functools.partial(_pass1_kernel, nblk=nblk),
        grid=(nblk,),
        in_specs=[
            pl.BlockSpec((1, 1, _BLK), lambda i: (i, 0, 0)),
            pl.BlockSpec((_BLK, f), lambda i: (i, 0)),
        ],
        out_specs=[
            pl.BlockSpec(memory_space=pltpu.SMEM),
            pl.BlockSpec((1, f), lambda i: (0, 0)),
        ],
        out_shape=[
            jax.ShapeDtypeStruct((1, 2), jnp.float32),
            jax.ShapeDtypeStruct((1, f), jnp.float32),
        ],
        scratch_shapes=[pltpu.SMEM((1, 2), jnp.float32)],
    )(tgt3, feat)


def _combine_kernel(cen_ref, s_ref, counts_ref, scal_ref, out_ref, *, batch):
    cen = cen_ref[...]
    c = cen.shape[0]
    s_sum = s_ref[0] + s_ref[1]
    dot_sum = jnp.sum(cen * s_sum[:c, :])
    c2 = jnp.sum(cen * cen, axis=1, keepdims=True)
    cterm = jax.lax.dot_general(
        counts_ref[...], c2,
        (((1,), (0,)), ((), ())),
        precision=jax.lax.Precision.HIGHEST,
        preferred_element_type=jnp.float32,
    )[0, 0]
    soft = scal_ref[0, 0]
    f2 = scal_ref[0, 1]
    center = cterm - 2.0 * dot_sum + f2
    out_ref[0, 0] = soft / batch + _LAMDA * center / 2.0 / batch


def _tc_combine(centers, s_partials, counts, scalars, batch):
    c, f = centers.shape
    return pl.pallas_call(
        functools.partial(_combine_kernel, batch=batch),
        in_specs=[
            pl.BlockSpec((c, f), lambda: (0, 0)),
            pl.BlockSpec((2, _SROWS, f), lambda: (0, 0, 0)),
            pl.BlockSpec((1, c), lambda: (0, 0)),
            pl.BlockSpec(memory_space=pltpu.SMEM),
        ],
        out_specs=pl.BlockSpec(memory_space=pltpu.SMEM),
        out_shape=jax.ShapeDtypeStruct((1, 1), jnp.float32),
    )(centers, s_partials, counts, scalars)


def kernel(feat, target, centers):
    batch = feat.shape[0]
    tgt = target.astype(jnp.int32)
    s_partials = _sc_segment_sum(feat, tgt)
    scalars, counts = _tc_pass1(feat, tgt)
    out = _tc_combine(centers, s_partials, counts, scalars, batch)
    return out[0, 0]
